# baseline (speedup 1.0000x reference)
import functools

import jax
import jax.numpy as jnp
from jax import lax
from jax.experimental import pallas as pl
from jax.experimental.pallas import tpu as pltpu
from jax.experimental.pallas import tpu_sc as plsc

D_MODEL = 1024
N_EXPERTS = 8
CAPACITY = 512
D_FF = 2048
EPAD = 128

_SC_CORES = 2
_SC_SUBCORES = 16
_SC_WORKERS = _SC_CORES * _SC_SUBCORES
_GROWS = 64


def _sc_gather(table, idx):
    b = idx.shape[0]
    d = table.shape[1]
    chunks = b // (_SC_WORKERS * _GROWS)
    mesh = plsc.VectorSubcoreMesh(core_axis_name="c", subcore_axis_name="s")

    @functools.partial(
        pl.kernel,
        mesh=mesh,
        out_type=jax.ShapeDtypeStruct((b, d), table.dtype),
        scratch_types=[
            pltpu.VMEM((_GROWS,), jnp.int32),
            pltpu.VMEM((_GROWS, d), table.dtype),
            pltpu.SemaphoreType.DMA,
        ],
    )
    def k(table_hbm, idx_hbm, out_hbm, idx_v, rows_v, sem):
        wid = lax.axis_index("s") * _SC_CORES + lax.axis_index("c")
        for c in range(chunks):
            base = (wid * chunks + c) * _GROWS
            pltpu.sync_copy(idx_hbm.at[pl.ds(base, _GROWS)], idx_v)
            pltpu.async_copy(table_hbm.at[idx_v], rows_v, sem).wait()
            pltpu.sync_copy(rows_v, out_hbm.at[pl.ds(base, _GROWS)])

    return k(table, idx)


def _router_kernel(h_ref, rw_ref, tslot_ref, cidx_ref, cw_ref, scale_ref, *, T):
    f32 = jnp.float32
    h = h_ref[...]
    logits = jnp.dot(h, rw_ref[...], preferred_element_type=f32)
    lane = lax.broadcasted_iota(jnp.int32, (T, EPAD), 1)
    lm = jnp.where(lane < N_EXPERTS, logits, -jnp.inf)

    mx = jnp.max(lm, axis=1, keepdims=True)
    ex = jnp.exp(lm - mx)
    probs = ex / jnp.sum(ex, axis=1, keepdims=True)

    i1 = jnp.min(jnp.where(lm >= mx, lane, EPAD), axis=1, keepdims=True)
    l2 = jnp.where(lane == i1, -jnp.inf, lm)
    m2 = jnp.max(l2, axis=1, keepdims=True)
    i2 = jnp.min(jnp.where(l2 >= m2, lane, EPAD), axis=1, keepdims=True)
    topm = (lane == i1) | (lane == i2)

    sT = jnp.transpose(lm)
    mT = jnp.transpose(topm.astype(f32))

    tok_row = lax.broadcasted_iota(jnp.int32, (1, T), 1)
    RB = 256
    rank_fl = jnp.zeros((T, EPAD), f32)
    kept_fl = jnp.zeros((T, EPAD), f32)
    for e in range(N_EXPERTS):
        s_row = sT[e : e + 1, :]
        m_row = mT[e : e + 1, :] > 0
        parts = []
        for bidx in range(T // RB):
            s_col = lm[bidx * RB : (bidx + 1) * RB, e : e + 1]
            tok_col = lax.broadcasted_iota(jnp.int32, (RB, 1), 0) + bidx * RB
            beats = (s_row > s_col) | ((s_row == s_col) & (tok_row < tok_col))
            cnt = jnp.sum(
                jnp.where(beats & m_row, 1.0, 0.0), axis=1, keepdims=True
            )
            parts.append(cnt)
        rank_col = jnp.concatenate(parts, axis=0)
        kept_col = topm[:, e : e + 1] & (rank_col < CAPACITY)

        c_iota = lax.broadcasted_iota(jnp.int32, (T, CAPACITY), 1)
        tok_colf = lax.broadcasted_iota(jnp.int32, (T, 1), 0).astype(f32)
        onehot = (rank_col.astype(jnp.int32) == c_iota) & kept_col
        tslot_ref[e : e + 1, :] = jnp.sum(
            jnp.where(onehot, tok_colf, 0.0), axis=0, keepdims=True
        ).astype(jnp.int32)

        sel = lane == e
        rank_fl = rank_fl + jnp.where(sel, rank_col, 0.0)
        kept_fl = kept_fl + jnp.where(sel, kept_col.astype(f32), 0.0)

    def pick(x, i):
        return jnp.sum(jnp.where(lane == i, x, 0.0), axis=1, keepdims=True)

    k1 = pick(kept_fl, i1) > 0
    k2 = pick(kept_fl, i2) > 0
    r1 = pick(rank_fl, i1).astype(jnp.int32)
    r2 = pick(rank_fl, i2).astype(jnp.int32)
    w0 = jnp.where(k1, pick(probs, i1), 0.0)
    w1 = jnp.where(k2, pick(probs, i2), 0.0)
    cidx_ref[:, 0:1] = jnp.where(k1, i1 * CAPACITY + r1, 0)
    cidx_ref[:, 1:2] = jnp.where(k2, i2 * CAPACITY + r2, 0)
    cw_ref[:, 0:1] = w0
    cw_ref[:, 1:2] = w1
    scale_ref[...] = 1.0 - w0 - w1


def _ffn_kernel(x_ref, w1_ref, w3_ref, w2_ref, o_ref):
    bf16 = jnp.bfloat16
    f = pl.program_id(1)
    x = x_ref[0].astype(bf16)
    a = jnp.dot(x, w1_ref[0].astype(bf16), preferred_element_type=jnp.float32)
    b = jnp.dot(x, w3_ref[0].astype(bf16), preferred_element_type=jnp.float32)
    u = (a * lax.logistic(a) * b).astype(bf16)
    part = jnp.dot(u, w2_ref[0].astype(bf16), preferred_element_type=jnp.float32)

    @pl.when(f == 0)
    def _():
        o_ref[0] = part

    @pl.when(f > 0)
    def _():
        o_ref[0] = o_ref[0] + part


def _combine_norm_kernel(h_ref, g_ref, cw_ref, sc_ref, lnw_ref, o_ref):
    h = h_ref[...]
    g0 = g_ref[:, :D_MODEL]
    g1 = g_ref[:, D_MODEL:]
    hn = h * sc_ref[...] + g0 * cw_ref[:, 0:1] + g1 * cw_ref[:, 1:2]
    var = jnp.mean(hn * hn, axis=1, keepdims=True)
    o_ref[...] = hn * lax.rsqrt(var + 1e-6) * lnw_ref[...]


def _logits_kernel(hn_ref, e_ref, o_ref):
    o_ref[...] = lax.dot_general(
        hn_ref[...].astype(jnp.bfloat16),
        e_ref[...].astype(jnp.bfloat16),
        (((1,), (1,)), ((), ())),
        preferred_element_type=jnp.float32,
    )


def kernel(ids, embed, router_w, w1, w3, w2, ln_w):
    T = ids.shape[0]
    V = embed.shape[0]

    h = _sc_gather(embed, ids.astype(jnp.int32))

    rw_pad = jnp.pad(router_w[0], ((0, 0), (0, EPAD - N_EXPERTS)))
    tslot, cidx, cw, scale = pl.pallas_call(
        functools.partial(_router_kernel, T=T),
        out_shape=(
            jax.ShapeDtypeStruct((N_EXPERTS, CAPACITY), jnp.int32),
            jax.ShapeDtypeStruct((T, 2), jnp.int32),
            jax.ShapeDtypeStruct((T, 2), jnp.float32),
            jax.ShapeDtypeStruct((T, 1), jnp.float32),
        ),
    )(h, rw_pad)

    xin = _sc_gather(h, tslot.reshape(-1))

    FB = 1024
    eo = pl.pallas_call(
        _ffn_kernel,
        grid=(N_EXPERTS, D_FF // FB),
        in_specs=[
            pl.BlockSpec((1, CAPACITY, D_MODEL), lambda e, f: (e, 0, 0)),
            pl.BlockSpec((1, D_MODEL, FB), lambda e, f: (e, 0, f)),
            pl.BlockSpec((1, D_MODEL, FB), lambda e, f: (e, 0, f)),
            pl.BlockSpec((1, FB, D_MODEL), lambda e, f: (e, f, 0)),
        ],
        out_specs=pl.BlockSpec((1, CAPACITY, D_MODEL), lambda e, f: (e, 0, 0)),
        out_shape=jax.ShapeDtypeStruct(
            (N_EXPERTS, CAPACITY, D_MODEL), jnp.float32
        ),
        compiler_params=pltpu.CompilerParams(
            dimension_semantics=("arbitrary", "arbitrary")
        ),
    )(xin.reshape(N_EXPERTS, CAPACITY, D_MODEL), w1, w3, w2)

    g = _sc_gather(eo.reshape(N_EXPERTS * CAPACITY, D_MODEL), cidx.reshape(-1))

    hn = pl.pallas_call(
        _combine_norm_kernel,
        out_shape=jax.ShapeDtypeStruct((T, D_MODEL), jnp.float32),
    )(h, g.reshape(T, 2 * D_MODEL), cw, scale, ln_w.reshape(1, D_MODEL))

    VB = 1280
    logits = pl.pallas_call(
        _logits_kernel,
        grid=(V // VB,),
        in_specs=[
            pl.BlockSpec((T, D_MODEL), lambda v: (0, 0)),
            pl.BlockSpec((VB, D_MODEL), lambda v: (v, 0)),
        ],
        out_specs=pl.BlockSpec((T, VB), lambda v: (0, v)),
        out_shape=jax.ShapeDtypeStruct((T, V), jnp.float32),
    )(hn, embed)
    return logits


# speedup vs baseline: 1.0051x; 1.0051x over previous
import functools

import jax
import jax.numpy as jnp
from jax import lax
from jax.experimental import pallas as pl
from jax.experimental.pallas import tpu as pltpu
from jax.experimental.pallas import tpu_sc as plsc

D_MODEL = 1024
N_EXPERTS = 8
CAPACITY = 512
D_FF = 2048
EPAD = 128

_SC_CORES = 2
_SC_SUBCORES = 16
_SC_WORKERS = _SC_CORES * _SC_SUBCORES
_GROWS = 64


def _sc_gather(table, idx):
    b = idx.shape[0]
    d = table.shape[1]
    chunks = b // (_SC_WORKERS * _GROWS)
    mesh = plsc.VectorSubcoreMesh(core_axis_name="c", subcore_axis_name="s")

    @functools.partial(
        pl.kernel,
        mesh=mesh,
        out_type=jax.ShapeDtypeStruct((b, d), table.dtype),
        scratch_types=[
            pltpu.VMEM((_GROWS,), jnp.int32),
            pltpu.VMEM((_GROWS, d), table.dtype),
            pltpu.SemaphoreType.DMA,
        ],
    )
    def k(table_hbm, idx_hbm, out_hbm, idx_v, rows_v, sem):
        wid = lax.axis_index("s") * _SC_CORES + lax.axis_index("c")
        for c in range(chunks):
            base = (wid * chunks + c) * _GROWS
            pltpu.sync_copy(idx_hbm.at[pl.ds(base, _GROWS)], idx_v)
            pltpu.async_copy(table_hbm.at[idx_v], rows_v, sem).wait()
            pltpu.sync_copy(rows_v, out_hbm.at[pl.ds(base, _GROWS)])

    return k(table, idx)


def _router_kernel(h_ref, rw_ref, tslot_ref, cidx_ref, cw_ref, scale_ref, *, T):
    f32 = jnp.float32
    i32 = jnp.int32
    h = h_ref[...]
    logits = jnp.dot(h, rw_ref[...], preferred_element_type=f32)
    lane = lax.broadcasted_iota(i32, (T, EPAD), 1)
    lm = jnp.where(lane < N_EXPERTS, logits, -jnp.inf)

    mx = jnp.max(lm, axis=1, keepdims=True)
    ex = jnp.exp(lm - mx)
    probs = ex / jnp.sum(ex, axis=1, keepdims=True)

    i1 = jnp.min(jnp.where(lm >= mx, lane, EPAD), axis=1, keepdims=True)
    l2 = jnp.where(lane == i1, -jnp.inf, lm)
    m2 = jnp.max(l2, axis=1, keepdims=True)
    i2 = jnp.min(jnp.where(l2 >= m2, lane, EPAD), axis=1, keepdims=True)
    topm = (lane == i1) | (lane == i2)
    topf = topm.astype(f32)

    bits = lax.bitcast_convert_type(logits + 0.0, i32)
    key = jnp.where(bits < 0, jnp.bitwise_xor(~bits, i32(-(2**31))), bits)
    key = jnp.where(topm & (lane < N_EXPERTS), key, i32(-(2**31)))

    def bstep(i, v):
        cand = v + lax.shift_left(i32(1), i32(31) - i)
        cnt = jnp.sum((key >= cand).astype(f32), axis=0, keepdims=True)
        return jnp.where(cnt >= CAPACITY, cand, v)

    v = lax.fori_loop(0, 32, bstep, jnp.full((1, EPAD), -(2**31), i32))

    above = (key > v).astype(f32)
    at = (key == v).astype(f32)
    n_above = jnp.sum(above, axis=0, keepdims=True)

    BT = 256
    tri = (
        lax.broadcasted_iota(i32, (BT, BT), 1)
        < lax.broadcasted_iota(i32, (BT, BT), 0)
    ).astype(jnp.bfloat16)

    def prefix_excl(x):
        xb = x.astype(jnp.bfloat16)
        parts = []
        tot = jnp.zeros((1, EPAD), f32)
        for b in range(T // BT):
            blk = xb[b * BT : (b + 1) * BT]
            parts.append(jnp.dot(tri, blk, preferred_element_type=f32) + tot)
            tot = tot + jnp.sum(
                x[b * BT : (b + 1) * BT], axis=0, keepdims=True
            )
        return jnp.concatenate(parts, axis=0)

    tie_rank = prefix_excl(at)
    kept = (above + at * (tie_rank < (CAPACITY - n_above)).astype(f32)) * topf
    slot = prefix_excl(kept)

    slot_i = slot.astype(i32)
    kept_b = kept > 0
    cap_iota = lax.broadcasted_iota(i32, (T, CAPACITY), 1)
    tok_colf = lax.broadcasted_iota(i32, (T, 1), 0).astype(f32)
    for e in range(N_EXPERTS):
        onehot = (slot_i[:, e : e + 1] == cap_iota) & kept_b[:, e : e + 1]
        tslot_ref[e : e + 1, :] = jnp.sum(
            jnp.where(onehot, tok_colf, 0.0), axis=0, keepdims=True
        ).astype(i32)

    def pick(x, i):
        return jnp.sum(jnp.where(lane == i, x, 0.0), axis=1, keepdims=True)

    k1 = pick(kept, i1) > 0
    k2 = pick(kept, i2) > 0
    s1 = pick(slot, i1).astype(i32)
    s2 = pick(slot, i2).astype(i32)
    w0 = jnp.where(k1, pick(probs, i1), 0.0)
    w1 = jnp.where(k2, pick(probs, i2), 0.0)
    cidx_ref[:, 0:1] = jnp.where(k1, i1 * CAPACITY + s1, 0)
    cidx_ref[:, 1:2] = jnp.where(k2, i2 * CAPACITY + s2, 0)
    cw_ref[:, 0:1] = w0
    cw_ref[:, 1:2] = w1
    scale_ref[...] = 1.0 - w0 - w1


def _ffn_kernel(x_ref, w1_ref, w3_ref, w2_ref, o_ref):
    bf16 = jnp.bfloat16
    f = pl.program_id(1)
    x = x_ref[0].astype(bf16)
    a = jnp.dot(x, w1_ref[0].astype(bf16), preferred_element_type=jnp.float32)
    b = jnp.dot(x, w3_ref[0].astype(bf16), preferred_element_type=jnp.float32)
    u = (a * lax.logistic(a) * b).astype(bf16)
    part = jnp.dot(u, w2_ref[0].astype(bf16), preferred_element_type=jnp.float32)

    @pl.when(f == 0)
    def _():
        o_ref[0] = part

    @pl.when(f > 0)
    def _():
        o_ref[0] = o_ref[0] + part


def _logits_kernel(h_ref, g_ref, cw_ref, sc_ref, lnw_ref, e_ref, o_ref, hn_ref):
    @pl.when(pl.program_id(0) == 0)
    def _():
        h = h_ref[...]
        g0 = g_ref[:, :D_MODEL]
        g1 = g_ref[:, D_MODEL:]
        hn = h * sc_ref[...] + g0 * cw_ref[:, 0:1] + g1 * cw_ref[:, 1:2]
        var = jnp.mean(hn * hn, axis=1, keepdims=True)
        hn_ref[...] = (hn * lax.rsqrt(var + 1e-6) * lnw_ref[...]).astype(
            jnp.bfloat16
        )

    o_ref[...] = lax.dot_general(
        hn_ref[...],
        e_ref[...].astype(jnp.bfloat16),
        (((1,), (1,)), ((), ())),
        preferred_element_type=jnp.float32,
    )


def kernel(ids, embed, router_w, w1, w3, w2, ln_w):
    T = ids.shape[0]
    V = embed.shape[0]

    h = _sc_gather(embed, ids.astype(jnp.int32))

    rw_pad = jnp.pad(router_w[0], ((0, 0), (0, EPAD - N_EXPERTS)))
    tslot, cidx, cw, scale = pl.pallas_call(
        functools.partial(_router_kernel, T=T),
        out_shape=(
            jax.ShapeDtypeStruct((N_EXPERTS, CAPACITY), jnp.int32),
            jax.ShapeDtypeStruct((T, 2), jnp.int32),
            jax.ShapeDtypeStruct((T, 2), jnp.float32),
            jax.ShapeDtypeStruct((T, 1), jnp.float32),
        ),
    )(h, rw_pad)

    xin = _sc_gather(h, tslot.reshape(-1))

    FB = 1024
    eo = pl.pallas_call(
        _ffn_kernel,
        grid=(N_EXPERTS, D_FF // FB),
        in_specs=[
            pl.BlockSpec((1, CAPACITY, D_MODEL), lambda e, f: (e, 0, 0)),
            pl.BlockSpec((1, D_MODEL, FB), lambda e, f: (e, 0, f)),
            pl.BlockSpec((1, D_MODEL, FB), lambda e, f: (e, 0, f)),
            pl.BlockSpec((1, FB, D_MODEL), lambda e, f: (e, f, 0)),
        ],
        out_specs=pl.BlockSpec((1, CAPACITY, D_MODEL), lambda e, f: (e, 0, 0)),
        out_shape=jax.ShapeDtypeStruct(
            (N_EXPERTS, CAPACITY, D_MODEL), jnp.float32
        ),
        compiler_params=pltpu.CompilerParams(
            dimension_semantics=("arbitrary", "arbitrary")
        ),
    )(xin.reshape(N_EXPERTS, CAPACITY, D_MODEL), w1, w3, w2)

    g = _sc_gather(eo.reshape(N_EXPERTS * CAPACITY, D_MODEL), cidx.reshape(-1))

    VB = 640
    logits = pl.pallas_call(
        _logits_kernel,
        grid=(V // VB,),
        in_specs=[
            pl.BlockSpec((T, D_MODEL), lambda v: (0, 0)),
            pl.BlockSpec((T, 2 * D_MODEL), lambda v: (0, 0)),
            pl.BlockSpec((T, 2), lambda v: (0, 0)),
            pl.BlockSpec((T, 1), lambda v: (0, 0)),
            pl.BlockSpec((1, D_MODEL), lambda v: (0, 0)),
            pl.BlockSpec((VB, D_MODEL), lambda v: (v, 0)),
        ],
        out_specs=pl.BlockSpec((T, VB), lambda v: (0, v)),
        out_shape=jax.ShapeDtypeStruct((T, V), jnp.float32),
        scratch_shapes=[pltpu.VMEM((T, D_MODEL), jnp.bfloat16)],
    )(h, g.reshape(T, 2 * D_MODEL), cw, scale, ln_w.reshape(1, D_MODEL), embed)
    return logits
